# Initial kernel scaffold; baseline (speedup 1.0000x reference)
import functools

import jax
import jax.numpy as jnp
from jax import lax
from jax.experimental import pallas as pl
from jax.experimental.pallas import tpu as pltpu
from jax.experimental.pallas import tpu_sc as plsc


def _build(N, D, E, NC, NT, L):
  K = 128
  chalf = D // NC
  rpt = -(-(N + 1) // (NT * K)) * K
  n_pad = NT * rpt
  CH = -(-(E + N) // (NT * K))
  if CH % 2:
    CH += 1
  etot = NT * CH * K
  nrow_ch = rpt // K
  vpr = chalf // L

  mesh = plsc.VectorSubcoreMesh(core_axis_name="c", subcore_axis_name="s",
                                num_cores=NC, num_subcores=NT)

  @functools.partial(
      pl.kernel,
      out_type=(
          jax.ShapeDtypeStruct((n_pad, D), jnp.float32),
          jax.ShapeDtypeStruct((n_pad, D), jnp.float32),
      ),
      mesh=mesh,
      scratch_types=[
          pltpu.VMEM((CH, K), jnp.int32),
          pltpu.VMEM((CH, K), jnp.int32),
          pltpu.VMEM((K, chalf), jnp.float32),
          pltpu.VMEM((K, chalf), jnp.float32),
          pltpu.VMEM((K, chalf), jnp.float32),
          pltpu.VMEM((K, chalf), jnp.float32),
          pltpu.VMEM((K, chalf), jnp.float32),
          pltpu.VMEM((rpt,), jnp.float32),
          pltpu.VMEM((rpt,), jnp.float32),
          pltpu.VMEM((rpt,), jnp.float32),
          pltpu.VMEM_SHARED((n_pad, chalf), jnp.float32),
          pltpu.SemaphoreType.DMA,
          pltpu.SemaphoreType.DMA,
      ],
  )
  def lightgcn(cols_ref, rows_ref, x0_ref, a_ref, y_ref,
               cidx, ridx, g0, g1, sbuf, abuf, zbuf, degp, disb, dis2b,
               s_acc, sem0, sem1):
    c = lax.axis_index("c")
    s = lax.axis_index("s")
    base = s * rpt
    coff = c * chalf
    colsl = pl.ds(coff, chalf)

    @pl.loop(0, K)
    def _(r):
      for k in range(vpr):
        zbuf[r, pl.ds(L * k, L)] = jnp.zeros((L,), jnp.float32)

    pltpu.sync_copy(rows_ref.at[s], ridx)

    for i in range(rpt // L):
      degp[pl.ds(L * i, L)] = jnp.zeros((L,), jnp.float32)
    ones = jnp.ones((L,), jnp.float32)
    for t in range(NT):
      pltpu.sync_copy(cols_ref.at[t], cidx)

      @pl.loop(0, CH)
      def _(j):
        for k in range(K // L):
          idx = cidx[j, pl.ds(L * k, L)]
          loc = idx - base
          m = (loc >= 0) & (loc < rpt)
          locs = jnp.where(m, loc, 0)
          plsc.addupdate_scatter(degp, [locs], ones, mask=m)
    pltpu.sync_copy(cols_ref.at[s], cidx)

    for i in range(rpt // L):
      sl = pl.ds(L * i, L)
      d = degp[sl]
      bits = plsc.bitcast(d, jnp.int32)
      y = plsc.bitcast(jnp.int32(0x5F3759DF) - (bits >> 1), jnp.float32)
      for _ in range(3):
        y = y * (1.5 - 0.5 * d * y * y)
      dis = jnp.where(d > 0.0, y, 0.0)
      disb[sl] = dis
      dis2b[sl] = dis * dis

    for i in range(nrow_ch):
      r0 = base + K * i
      rowsl = pl.ds(r0, K)
      pltpu.sync_copy(x0_ref.at[rowsl, colsl], sbuf)
      pltpu.sync_copy(sbuf, a_ref.at[rowsl, colsl])

      @pl.loop(0, K)
      def _(r):
        dv = disb[K * i + r]
        for k in range(vpr):
          sl = pl.ds(L * k, L)
          sbuf[r, sl] = sbuf[r, sl] * dv

      pltpu.sync_copy(sbuf, y_ref.at[rowsl, colsl])
      pltpu.sync_copy(zbuf, s_acc.at[rowsl])
    plsc.subcore_barrier()

    def gsrc(j):
      return y_ref.at[cidx.at[j], colsl]

    for layer in range(3):
      last = layer == 2

      pltpu.async_copy(gsrc(0), g0, sem0)

      @pl.loop(0, CH // 2 - 1)
      def _(it):
        j = 2 * it
        pltpu.async_copy(gsrc(j + 1), g1, sem1)
        pltpu.make_async_copy(gsrc(j), g0, sem0).wait()
        pltpu.sync_copy(g0, s_acc.at[ridx.at[j]], add=True)
        pltpu.async_copy(gsrc(j + 2), g0, sem0)
        pltpu.make_async_copy(gsrc(j + 1), g1, sem1).wait()
        pltpu.sync_copy(g1, s_acc.at[ridx.at[j + 1]], add=True)

      jl = CH - 2
      pltpu.make_async_copy(gsrc(jl), g0, sem0).wait()
      pltpu.sync_copy(g0, s_acc.at[ridx.at[jl]], add=True)
      pltpu.async_copy(gsrc(jl + 1), g1, sem1)
      pltpu.make_async_copy(gsrc(jl + 1), g1, sem1).wait()
      pltpu.sync_copy(g1, s_acc.at[ridx.at[jl + 1]], add=True)
      plsc.subcore_barrier()

      for i in range(nrow_ch):
        r0 = base + K * i
        rowsl = pl.ds(r0, K)
        pltpu.sync_copy(s_acc.at[rowsl], sbuf)
        if not last:
          pltpu.sync_copy(zbuf, s_acc.at[rowsl])
        pltpu.sync_copy(a_ref.at[rowsl, colsl], abuf)

        @pl.loop(0, K)
        def _(r):
          dv = disb[K * i + r]
          d2 = dis2b[K * i + r]
          for k in range(vpr):
            sl = pl.ds(L * k, L)
            sv = sbuf[r, sl]
            av = abuf[r, sl] + dv * sv
            if last:
              av = av * 0.25
            abuf[r, sl] = av
            if not last:
              sbuf[r, sl] = d2 * sv

        pltpu.sync_copy(abuf, a_ref.at[rowsl, colsl])
        if not last:
          pltpu.sync_copy(sbuf, y_ref.at[rowsl, colsl])
      plsc.subcore_barrier()

  return lightgcn, CH, K, rpt, n_pad, etot


def kernel(edge_index, embedding_weight):
  N, D = embedding_weight.shape
  E = edge_index.shape[1]
  info = plsc.get_sparse_core_info()
  NC, NT, L = info.num_cores, info.num_subcores, info.num_lanes
  fn, CH, K, rpt, n_pad, etot = _build(N, D, E, NC, NT, L)

  loop = jnp.arange(N, dtype=jnp.int32)
  npad_e = etot - E - N
  trash = jnp.full((npad_e,), N, jnp.int32)
  row = jnp.concatenate([edge_index[0].astype(jnp.int32), loop, trash])
  col = jnp.concatenate([edge_index[1].astype(jnp.int32), loop, trash])
  cols3 = col.reshape(NT, CH, K)
  rows3 = row.reshape(NT, CH, K)
  x0p = jnp.zeros((n_pad, D), jnp.float32).at[:N].set(embedding_weight)
  a, _ = fn(cols3, rows3, x0p)
  return a[:N]


# speedup vs baseline: 7.1627x; 7.1627x over previous
import functools

import jax
import jax.numpy as jnp
from jax import lax
from jax.experimental import pallas as pl
from jax.experimental.pallas import tpu as pltpu
from jax.experimental.pallas import tpu_sc as plsc

_KE = 64
_CHB = 24
_RC = 64


def _build(N, D, E, NC, NT, L):
  chalf = D // NC
  rpt = -(-(N + 1) // (NT * _RC)) * _RC
  n_pad = NT * rpt
  nrow_ch = rpt // _RC
  vpr = chalf // L
  CH = -(-(E + N) // (NT * _KE))
  CH = -(-CH // _CHB) * _CHB
  nblk = CH // _CHB
  etot = NT * CH * _KE

  mesh = plsc.VectorSubcoreMesh(core_axis_name="c", subcore_axis_name="s",
                                num_cores=NC, num_subcores=NT)

  @functools.partial(
      pl.kernel,
      out_type=(
          jax.ShapeDtypeStruct((n_pad, D), jnp.float32),
          jax.ShapeDtypeStruct((n_pad, D), jnp.float32),
      ),
      mesh=mesh,
      compiler_params=pltpu.CompilerParams(needs_layout_passes=False),
      scratch_types=[
          pltpu.VMEM((_CHB, _KE), jnp.int32),
          pltpu.VMEM((_CHB, _KE), jnp.int32),
          pltpu.VMEM((_KE, chalf), jnp.float32),
          pltpu.VMEM((_KE, chalf), jnp.float32),
          pltpu.VMEM((_RC, chalf), jnp.float32),
          pltpu.VMEM((_RC, chalf), jnp.float32),
          pltpu.VMEM((_RC, chalf), jnp.float32),
          pltpu.VMEM((rpt,), jnp.float32),
          pltpu.VMEM((rpt,), jnp.float32),
          pltpu.VMEM((rpt,), jnp.float32),
          pltpu.VMEM_SHARED((n_pad, chalf), jnp.float32),
          pltpu.SemaphoreType.DMA,
          pltpu.SemaphoreType.DMA,
      ],
  )
  def lightgcn(cols_ref, rows_ref, x0_ref, a_ref, y_ref,
               cidx, ridx, g0, g1, sbuf, abuf, zbuf, degp, disb, dis2b,
               s_acc, sem0, sem1):
    c = lax.axis_index("c")
    s = lax.axis_index("s")
    base = pl.multiple_of(s * rpt, _RC)
    coff = pl.multiple_of(c * chalf, chalf)
    colsl = pl.ds(coff, chalf)

    @pl.loop(0, _RC)
    def _(r):
      for k in range(vpr):
        zbuf[r, pl.ds(L * k, L)] = jnp.zeros((L,), jnp.float32)

    @pl.loop(0, rpt // L)
    def _(i):
      degp[pl.ds(pl.multiple_of(L * i, L), L)] = jnp.zeros((L,), jnp.float32)

    ones = jnp.ones((L,), jnp.float32)

    @pl.loop(0, NT * nblk)
    def _(tb):
      t = tb // nblk
      b = tb % nblk
      bsl = pl.ds(pl.multiple_of(b * _CHB, _CHB), _CHB)
      pltpu.sync_copy(cols_ref.at[t, bsl], cidx)

      @pl.loop(0, _CHB)
      def _(j):
        for k in range(_KE // L):
          idx = cidx[j, pl.ds(L * k, L)]
          loc = idx - base
          m = (loc >= 0) & (loc < rpt)
          locs = jnp.where(m, loc, 0)
          plsc.addupdate_scatter(degp, [locs], ones, mask=m)

    @pl.loop(0, rpt // L)
    def _(i):
      sl = pl.ds(pl.multiple_of(L * i, L), L)
      d = degp[sl]
      bits = plsc.bitcast(d, jnp.int32)
      y = plsc.bitcast(jnp.int32(0x5F3759DF) - (bits >> 1), jnp.float32)
      for _ in range(3):
        y = y * (1.5 - 0.5 * d * y * y)
      dis = jnp.where(d > 0.0, y, 0.0)
      disb[sl] = dis
      dis2b[sl] = dis * dis

    @pl.loop(0, nrow_ch)
    def _(i):
      rowsl = pl.ds(pl.multiple_of(base + _RC * i, _RC), _RC)
      pltpu.sync_copy(x0_ref.at[rowsl, colsl], sbuf)
      pltpu.sync_copy(sbuf, a_ref.at[rowsl, colsl])

      @pl.loop(0, _RC)
      def _(r):
        dv = plsc.load_gather(disb, [jnp.full((L,), _RC * i + r, jnp.int32)])
        for k in range(vpr):
          sl = pl.ds(L * k, L)
          sbuf[r, sl] = sbuf[r, sl] * dv

      pltpu.sync_copy(sbuf, y_ref.at[rowsl, colsl])
      pltpu.sync_copy(zbuf, s_acc.at[rowsl])

    plsc.subcore_barrier()

    def gsrc(j):
      return y_ref.at[cidx.at[j], colsl]

    for layer in range(3):
      last = layer == 2

      @pl.loop(0, nblk)
      def _(b):
        bsl = pl.ds(pl.multiple_of(b * _CHB, _CHB), _CHB)
        pltpu.sync_copy(cols_ref.at[s, bsl], cidx)
        pltpu.sync_copy(rows_ref.at[s, bsl], ridx)
        pltpu.async_copy(gsrc(0), g0, sem0)

        @pl.loop(0, _CHB // 2 - 1)
        def _(it):
          j = 2 * it
          pltpu.async_copy(gsrc(j + 1), g1, sem1)
          pltpu.make_async_copy(gsrc(j), g0, sem0).wait()
          pltpu.sync_copy(g0, s_acc.at[ridx.at[j]], add=True)
          pltpu.async_copy(gsrc(j + 2), g0, sem0)
          pltpu.make_async_copy(gsrc(j + 1), g1, sem1).wait()
          pltpu.sync_copy(g1, s_acc.at[ridx.at[j + 1]], add=True)

        jl = _CHB - 2
        pltpu.make_async_copy(gsrc(jl), g0, sem0).wait()
        pltpu.sync_copy(g0, s_acc.at[ridx.at[jl]], add=True)
        pltpu.async_copy(gsrc(jl + 1), g1, sem1)
        pltpu.make_async_copy(gsrc(jl + 1), g1, sem1).wait()
        pltpu.sync_copy(g1, s_acc.at[ridx.at[jl + 1]], add=True)

      plsc.subcore_barrier()

      @pl.loop(0, nrow_ch)
      def _(i):
        rowsl = pl.ds(pl.multiple_of(base + _RC * i, _RC), _RC)
        pltpu.sync_copy(s_acc.at[rowsl], sbuf)
        if not last:
          pltpu.sync_copy(zbuf, s_acc.at[rowsl])
        pltpu.sync_copy(a_ref.at[rowsl, colsl], abuf)

        @pl.loop(0, _RC)
        def _(r):
          ri = jnp.full((L,), _RC * i + r, jnp.int32)
          dv = plsc.load_gather(disb, [ri])
          d2 = plsc.load_gather(dis2b, [ri])
          for k in range(vpr):
            sl = pl.ds(L * k, L)
            sv = sbuf[r, sl]
            av = abuf[r, sl] + dv * sv
            if last:
              av = av * 0.25
            abuf[r, sl] = av
            if not last:
              sbuf[r, sl] = d2 * sv

        pltpu.sync_copy(abuf, a_ref.at[rowsl, colsl])
        if not last:
          pltpu.sync_copy(sbuf, y_ref.at[rowsl, colsl])

      plsc.subcore_barrier()

  return lightgcn, CH, rpt, n_pad, etot


def kernel(edge_index, embedding_weight):
  N, D = embedding_weight.shape
  E = edge_index.shape[1]
  info = plsc.get_sparse_core_info()
  NC, NT, L = info.num_cores, info.num_subcores, info.num_lanes
  fn, CH, rpt, n_pad, etot = _build(N, D, E, NC, NT, L)

  loop = jnp.arange(N, dtype=jnp.int32)
  npad_e = etot - E - N
  trash = jnp.full((npad_e,), N, jnp.int32)
  row = jnp.concatenate([edge_index[0].astype(jnp.int32), loop, trash])
  col = jnp.concatenate([edge_index[1].astype(jnp.int32), loop, trash])
  cols3 = col.reshape(NT, CH, _KE)
  rows3 = row.reshape(NT, CH, _KE)
  x0p = jnp.zeros((n_pad, D), jnp.float32).at[:N].set(embedding_weight)
  a, _ = fn(cols3, rows3, x0p)
  return a[:N]
